# Initial kernel scaffold; baseline (speedup 1.0000x reference)
import functools

import jax
import jax.numpy as jnp
from jax import lax
from jax.experimental import pallas as pl
from jax.experimental.pallas import tpu as pltpu
from jax.experimental.pallas import tpu_sc as plsc

VOCAB = 100000
EMBED = 32
B = 16384
L = 50
N = B * L

NC = 2
NS = 16
NW = NC * NS
B_PER_W = N // NW
CHUNK = 128
K = 20
G = K * CHUNK
N_GROUPS = B_PER_W // G
ROWS_PER_W = B_PER_W // CHUNK


def _sc_gather(idx2d, table):
    mesh = plsc.VectorSubcoreMesh(core_axis_name="c", subcore_axis_name="s")

    @functools.partial(
        pl.kernel,
        mesh=mesh,
        out_type=jax.ShapeDtypeStruct((N, EMBED), jnp.float32),
        scratch_types=[
            pltpu.VMEM((K, CHUNK), jnp.int32),
            pltpu.VMEM((G, EMBED), jnp.float32),
            pltpu.SemaphoreType.DMA,
        ],
    )
    def body(idx_hbm, table_hbm, out_hbm, idx_v, rows_v, sem):
        wid = lax.axis_index("s") * NC + lax.axis_index("c")
        row0 = wid * ROWS_PER_W

        def group(g, carry):
            r = row0 + g * K
            pltpu.sync_copy(idx_hbm.at[pl.ds(r, K)], idx_v)
            copies = [
                pltpu.async_copy(
                    table_hbm.at[idx_v.at[j]],
                    rows_v.at[pl.ds(j * CHUNK, CHUNK)],
                    sem,
                )
                for j in range(K)
            ]
            for c in copies:
                c.wait()
            pltpu.sync_copy(rows_v, out_hbm.at[pl.ds(r * CHUNK, G)])
            return carry

        lax.fori_loop(0, N_GROUPS, group, 0)

    return body(idx2d, table)


def kernel(inputs, table):
    idx2d = inputs.astype(jnp.int32).reshape(N // CHUNK, CHUNK)
    return _sc_gather(idx2d, table)


# speedup vs baseline: 5.6253x; 5.6253x over previous
import functools

import jax
import jax.numpy as jnp
from jax import lax
from jax.experimental import pallas as pl
from jax.experimental.pallas import tpu as pltpu
from jax.experimental.pallas import tpu_sc as plsc

VOCAB = 100000
EMBED = 32
B = 16384
L = 50
N = B * L

NC = 2
NS = 16
NW = NC * NS
B_PER_W = N // NW
CHUNK = 128
K = 8
G = K * CHUNK
N_GROUPS = B_PER_W // G
ROWS_PER_W = B_PER_W // CHUNK


def _sc_gather(idx2d, table):
    mesh = plsc.VectorSubcoreMesh(core_axis_name="c", subcore_axis_name="s")

    @functools.partial(
        pl.kernel,
        mesh=mesh,
        compiler_params=pltpu.CompilerParams(use_tc_tiling_on_sc=False),
        out_type=jax.ShapeDtypeStruct((N, EMBED), jnp.float32),
        scratch_types=[
            pltpu.VMEM((K, CHUNK), jnp.int32),
            pltpu.VMEM((G, EMBED), jnp.float32),
            pltpu.SemaphoreType.DMA,
        ],
    )
    def body(idx_hbm, table_hbm, out_hbm, idx_v, rows_v, sem):
        wid = lax.axis_index("s") * NC + lax.axis_index("c")
        row0 = wid * ROWS_PER_W

        def group(g, carry):
            r = row0 + g * K
            pltpu.sync_copy(idx_hbm.at[pl.ds(r, K)], idx_v)
            copies = [
                pltpu.async_copy(
                    table_hbm.at[idx_v.at[j]],
                    rows_v.at[pl.ds(j * CHUNK, CHUNK)],
                    sem,
                )
                for j in range(K)
            ]
            for c in copies:
                c.wait()
            pltpu.sync_copy(rows_v, out_hbm.at[pl.ds(r * CHUNK, G)])
            return carry

        lax.fori_loop(0, N_GROUPS, group, 0)

    return body(idx2d, table)


def kernel(inputs, table):
    idx2d = inputs.astype(jnp.int32).reshape(N // CHUNK, CHUNK)
    return _sc_gather(idx2d, table)


# speedup vs baseline: 5.8650x; 1.0426x over previous
import functools

import jax
import jax.numpy as jnp
from jax import lax
from jax.experimental import pallas as pl
from jax.experimental.pallas import tpu as pltpu
from jax.experimental.pallas import tpu_sc as plsc

VOCAB = 100000
EMBED = 32
B = 16384
L = 50
N = B * L

NC = 2
NS = 16
NW = NC * NS
B_PER_W = N // NW
CHUNK = 128
K = 8
G = K * CHUNK
N_GROUPS = B_PER_W // G
ROWS_PER_W = B_PER_W // CHUNK
LAST = N_GROUPS - 1


def _sc_gather(idx2d, table):
    mesh = plsc.VectorSubcoreMesh(core_axis_name="c", subcore_axis_name="s")

    @functools.partial(
        pl.kernel,
        mesh=mesh,
        compiler_params=pltpu.CompilerParams(use_tc_tiling_on_sc=False),
        out_type=jax.ShapeDtypeStruct((N, EMBED), jnp.float32),
        scratch_types=[
            pltpu.VMEM((2 * K, CHUNK), jnp.int32),
            pltpu.VMEM((2 * G, EMBED), jnp.float32),
            pltpu.SemaphoreType.DMA,
            pltpu.SemaphoreType.DMA,
        ],
    )
    def body(idx_hbm, table_hbm, out_hbm, idx_v, rows_v, sem_g, sem_o):
        wid = lax.axis_index("s") * NC + lax.axis_index("c")
        row0 = wid * ROWS_PER_W

        def fire(buf):
            for j in range(K):
                pltpu.async_copy(
                    table_hbm.at[idx_v.at[buf * K + j]],
                    rows_v.at[pl.ds(buf * G + j * CHUNK, CHUNK)],
                    sem_g,
                )

        def step(g, cur, first):
            nxt = 1 - cur
            r = row0 + g * K

            @pl.when(g < LAST)
            def _():
                pltpu.sync_copy(
                    idx_hbm.at[pl.ds(r + K, K)],
                    idx_v.at[pl.ds(nxt * K, K)],
                )

            pltpu.make_async_copy(
                out_hbm.at[pl.ds(r * CHUNK, G)],
                rows_v.at[pl.ds(cur * G, G)],
                sem_g,
            ).wait()

            @pl.when(g < LAST)
            def _():
                if not first:
                    pltpu.make_async_copy(
                        rows_v.at[pl.ds(nxt * G, G)],
                        out_hbm.at[pl.ds((r - K) * CHUNK, G)],
                        sem_o,
                    ).wait()
                fire(nxt)

            pltpu.async_copy(
                rows_v.at[pl.ds(cur * G, G)],
                out_hbm.at[pl.ds(r * CHUNK, G)],
                sem_o,
            )

        pltpu.sync_copy(
            idx_hbm.at[pl.ds(row0, K)], idx_v.at[pl.ds(0, K)]
        )
        fire(0)
        step(0, 0, True)

        def pair(i, carry):
            g = 2 * i + 1
            step(g, 1, False)
            step(g + 1, 0, False)
            return carry

        lax.fori_loop(0, (N_GROUPS - 1) // 2, pair, 0)

        for (gg, buf) in ((LAST - 1, 1), (LAST, 0)):
            pltpu.make_async_copy(
                rows_v.at[pl.ds(buf * G, G)],
                out_hbm.at[pl.ds((row0 + gg * K) * CHUNK, G)],
                sem_o,
            ).wait()

    return body(idx2d, table)


def kernel(inputs, table):
    idx2d = inputs.astype(jnp.int32).reshape(N // CHUNK, CHUNK)
    return _sc_gather(idx2d, table)
